# baseline (speedup 1.0000x reference)
import functools

import jax
import jax.numpy as jnp
from jax import lax
from jax.experimental import pallas as pl
from jax.experimental.pallas import tpu as pltpu
from jax.experimental.pallas import tpu_sc as plsc

D = 1024
KGRP = 4
N_TOKENS = 8192
NW = 32
CHUNK = 16
NBUF = 4
SLICES = (2048, 2048, 2048, 1536, 512)


def _sc_gather(table, idx, n_rows):
    rows_per_w = n_rows // NW
    nchunk = rows_per_w // CHUNK
    nquad = nchunk // NBUF
    mesh = plsc.VectorSubcoreMesh(core_axis_name="c", subcore_axis_name="s")

    @functools.partial(
        pl.kernel,
        mesh=mesh,
        out_type=jax.ShapeDtypeStruct((n_rows, D), jnp.float32),
        scratch_types=[
            pltpu.VMEM((rows_per_w,), jnp.int32),
        ] + [pltpu.VMEM((CHUNK, D), jnp.float32)] * NBUF
          + [pltpu.SemaphoreType.DMA] * (2 * NBUF),
    )
    def gather_kernel(table_hbm, idx_hbm, out_hbm, idx_v, *scratch):
        bufs = scratch[:NBUF]
        gss = scratch[NBUF:2 * NBUF]
        oss = scratch[2 * NBUF:]
        wid = lax.axis_index("s") * 2 + lax.axis_index("c")
        base = wid * rows_per_w
        pltpu.sync_copy(idx_hbm.at[pl.ds(base, rows_per_w)], idx_v)

        def g_args(c, b):
            return (table_hbm.at[idx_v.at[pl.ds(c * CHUNK, CHUNK)]],
                    bufs[b], gss[b])

        def w_args(c, b):
            return (bufs[b], out_hbm.at[pl.ds(base + c * CHUNK, CHUNK)],
                    oss[b])

        for b in range(NBUF):
            pltpu.async_copy(*g_args(b, b))

        def body(q, carry):
            c0 = q * NBUF
            for b in range(NBUF):
                pltpu.make_async_copy(*g_args(c0 + b, b)).wait()
                pltpu.async_copy(*w_args(c0 + b, b))
            for b in range(NBUF):
                pltpu.make_async_copy(*w_args(c0 + b, b)).wait()
                pltpu.async_copy(*g_args(c0 + NBUF + b, b))
            return carry

        lax.fori_loop(0, nquad - 1, body, 0)

        c0 = (nquad - 1) * NBUF
        for b in range(NBUF):
            pltpu.make_async_copy(*g_args(c0 + b, b)).wait()
            pltpu.async_copy(*w_args(c0 + b, b))
        for b in range(NBUF):
            pltpu.make_async_copy(*w_args(c0 + b, b)).wait()

    return gather_kernel(table, idx)


_TM = 512


def _tc_matmul(planes, w, b2d, n_tok, tok_off, buf):
    tile_off = tok_off // _TM

    def body(*refs):
        a_ref, w_ref, b_ref = refs[0], refs[1], refs[2]
        o_ref = refs[-1]
        acc = jnp.broadcast_to(b_ref[...].astype(jnp.float32), (_TM, D))
        for k in range(KGRP):
            a = a_ref[k].astype(jnp.bfloat16)
            wk = w_ref[:, k * D:(k + 1) * D]
            acc = acc + lax.dot_general(
                a, wk, (((1,), (1,)), ((), ())),
                preferred_element_type=jnp.float32,
            )
        o_ref[...] = acc

    in_specs = [
        pl.BlockSpec((KGRP, _TM, D), lambda i: (0, i, 0)),
        pl.BlockSpec((D, KGRP * D), lambda i: (0, 0)),
        pl.BlockSpec((1, D), lambda i: (0, 0)),
    ]
    args = [planes, w, b2d]
    kwargs = {}
    if buf is not None:
        in_specs.append(pl.BlockSpec(memory_space=pl.ANY))
        args.append(buf)
        kwargs["input_output_aliases"] = {3: 0}

    return pl.pallas_call(
        body,
        grid=(n_tok // _TM,),
        in_specs=in_specs,
        out_specs=pl.BlockSpec((_TM, D), lambda i: (tile_off + i, 0)),
        out_shape=jax.ShapeDtypeStruct((N_TOKENS, D), jnp.float32),
        **kwargs,
    )(*args)


def kernel(x, emb_table, proj_w, proj_b):
    B, L, K = x.shape
    w_bf = proj_w.astype(jnp.bfloat16)
    b2d = proj_b.reshape(1, D)
    x_flat = x.reshape(N_TOKENS, K).astype(jnp.int32)
    buf = None
    tok_off = 0
    for n_tok in SLICES:
        sl = lax.slice_in_dim(x_flat, tok_off, tok_off + n_tok)
        idx = sl.T.reshape(-1)
        flat = _sc_gather(emb_table, idx, n_tok * KGRP)
        planes = flat.reshape(KGRP, n_tok, D)
        buf = _tc_matmul(planes, w_bf, b2d, n_tok, tok_off, buf)
        tok_off += n_tok
    return buf.reshape(B, L, D)


# speedup vs baseline: 1.0206x; 1.0206x over previous
import functools

import jax
import jax.numpy as jnp
from jax import lax
from jax.experimental import pallas as pl
from jax.experimental.pallas import tpu as pltpu
from jax.experimental.pallas import tpu_sc as plsc

D = 1024
KGRP = 4
N_TOKENS = 8192
NW = 32
CHUNK = 16
NBUF = 4
SLICES = (2048, 2048, 2048, 2048)


def _sc_gather(table, idx, n_rows):
    rows_per_w = n_rows // NW
    nchunk = rows_per_w // CHUNK
    nquad = nchunk // NBUF
    mesh = plsc.VectorSubcoreMesh(core_axis_name="c", subcore_axis_name="s")

    @functools.partial(
        pl.kernel,
        mesh=mesh,
        out_type=jax.ShapeDtypeStruct((n_rows, D), jnp.float32),
        scratch_types=[
            pltpu.VMEM((rows_per_w,), jnp.int32),
        ] + [pltpu.VMEM((CHUNK, D), jnp.float32)] * NBUF
          + [pltpu.SemaphoreType.DMA] * (2 * NBUF),
    )
    def gather_kernel(table_hbm, idx_hbm, out_hbm, idx_v, *scratch):
        bufs = scratch[:NBUF]
        gss = scratch[NBUF:2 * NBUF]
        oss = scratch[2 * NBUF:]
        wid = lax.axis_index("s") * 2 + lax.axis_index("c")
        base = wid * rows_per_w
        pltpu.sync_copy(idx_hbm.at[pl.ds(base, rows_per_w)], idx_v)

        def g_args(c, b):
            return (table_hbm.at[idx_v.at[pl.ds(c * CHUNK, CHUNK)]],
                    bufs[b], gss[b])

        def w_args(c, b):
            return (bufs[b], out_hbm.at[pl.ds(base + c * CHUNK, CHUNK)],
                    oss[b])

        for b in range(NBUF):
            pltpu.async_copy(*g_args(b, b))

        def body(q, carry):
            c0 = q * NBUF
            for b in range(NBUF):
                pltpu.make_async_copy(*g_args(c0 + b, b)).wait()
                pltpu.async_copy(*w_args(c0 + b, b))
            for b in range(NBUF):
                pltpu.make_async_copy(*w_args(c0 + b, b)).wait()
                pltpu.async_copy(*g_args(c0 + NBUF + b, b))
            return carry

        lax.fori_loop(0, nquad - 1, body, 0)

        c0 = (nquad - 1) * NBUF
        for b in range(NBUF):
            pltpu.make_async_copy(*g_args(c0 + b, b)).wait()
            pltpu.async_copy(*w_args(c0 + b, b))
        for b in range(NBUF):
            pltpu.make_async_copy(*w_args(c0 + b, b)).wait()

    return gather_kernel(table, idx)


_TM = 512


def _tc_matmul(planes, w, b2d, n_tok, tok_off, buf):
    tile_off = tok_off // _TM

    def body(*refs):
        a_ref, w_ref, b_ref = refs[0], refs[1], refs[2]
        o_ref = refs[-1]
        acc = jnp.broadcast_to(b_ref[...].astype(jnp.float32), (_TM, D))
        for k in range(KGRP):
            a = a_ref[k].astype(jnp.bfloat16)
            wk = w_ref[:, k * D:(k + 1) * D]
            acc = acc + lax.dot_general(
                a, wk, (((1,), (1,)), ((), ())),
                preferred_element_type=jnp.float32,
            )
        o_ref[...] = acc

    in_specs = [
        pl.BlockSpec((KGRP, _TM, D), lambda i: (0, i, 0)),
        pl.BlockSpec((D, KGRP * D), lambda i: (0, 0)),
        pl.BlockSpec((1, D), lambda i: (0, 0)),
    ]
    args = [planes, w, b2d]
    kwargs = {}
    if buf is not None:
        in_specs.append(pl.BlockSpec(memory_space=pl.ANY))
        args.append(buf)
        kwargs["input_output_aliases"] = {3: 0}

    return pl.pallas_call(
        body,
        grid=(n_tok // _TM,),
        in_specs=in_specs,
        out_specs=pl.BlockSpec((_TM, D), lambda i: (tile_off + i, 0)),
        out_shape=jax.ShapeDtypeStruct((N_TOKENS, D), jnp.float32),
        **kwargs,
    )(*args)


def kernel(x, emb_table, proj_w, proj_b):
    B, L, K = x.shape
    w_bf = proj_w.astype(jnp.bfloat16)
    b2d = proj_b.reshape(1, D)
    x_flat = x.reshape(N_TOKENS, K).astype(jnp.int32)
    buf = None
    tok_off = 0
    for n_tok in SLICES:
        sl = lax.slice_in_dim(x_flat, tok_off, tok_off + n_tok)
        idx = sl.T.reshape(-1)
        flat = _sc_gather(emb_table, idx, n_tok * KGRP)
        planes = flat.reshape(KGRP, n_tok, D)
        buf = _tc_matmul(planes, w_bf, b2d, n_tok, tok_off, buf)
        tok_off += n_tok
    return buf.reshape(B, L, D)
